# Initial kernel scaffold; baseline (speedup 1.0000x reference)
import jax
import jax.numpy as jnp
from jax.experimental import pallas as pl


def kernel(x, edge_index, W1, att_src1, att_dst1, b1, W2, att_src2, att_dst2, b2):
    raise NotImplementedError("write your pallas kernel here")


# speedup vs baseline: 22.8180x; 22.8180x over previous
import functools

import jax
import jax.numpy as jnp
from jax import lax
from jax.experimental import pallas as pl
from jax.experimental.pallas import tpu as pltpu
from jax.experimental.pallas import tpu_sc as plsc

N = 10000
NP = 10240
E = 320000
D = 128
HD = D // 2
NEG = 0.2

NC = 2
NS = 16
EPT = E // NS
K = 80
NCH = EPT // K
ZB = 80
ZSTRIDE = NP // NS


def _tc_pre_body(x_ref, w_ref, aw_s_ref, aw_d_ref,
                 hlo_ref, hhi_ref, asrc_ref, adst_ref):
    h = jnp.dot(x_ref[...], w_ref[...], preferred_element_type=jnp.float32)
    hlo_ref[...] = h[:, :HD]
    hhi_ref[...] = h[:, HD:]
    a_s = jnp.dot(h, aw_s_ref[...], preferred_element_type=jnp.float32)
    a_d = jnp.dot(h, aw_d_ref[...], preferred_element_type=jnp.float32)
    amax = jnp.max(a_s)
    asrc_ref[...] = jnp.concatenate(
        [a_s, jnp.full((16, 1), amax, jnp.float32)], axis=0)
    adst_ref[...] = a_d


def _tc_pre(x, w, aw_s, aw_d):
    return pl.pallas_call(
        _tc_pre_body,
        out_shape=[
            jax.ShapeDtypeStruct((N, HD), jnp.float32),
            jax.ShapeDtypeStruct((N, HD), jnp.float32),
            jax.ShapeDtypeStruct((N + 16, 1), jnp.float32),
            jax.ShapeDtypeStruct((N, 1), jnp.float32),
        ],
    )(x, w, aw_s, aw_d)


def _tc_mid_body(part_ref, wsum_ref, b_ref, w2_ref, aw_s_ref, aw_d_ref,
                 x1_ref, hlo_ref, hhi_ref, asrc_ref, adst_ref):
    unnorm = jnp.concatenate([part_ref[0, :N], part_ref[1, :N]], axis=1)
    wsum = wsum_ref[:N]
    x1 = unnorm / (wsum[:, None] + 1e-16) + b_ref[...][None, :]
    x1 = jnp.maximum(x1, 0.0)
    x1_ref[...] = x1
    h2 = jnp.dot(x1, w2_ref[...], preferred_element_type=jnp.float32)
    hlo_ref[...] = h2[:, :HD]
    hhi_ref[...] = h2[:, HD:]
    a_s = jnp.dot(h2, aw_s_ref[...], preferred_element_type=jnp.float32)
    a_d = jnp.dot(h2, aw_d_ref[...], preferred_element_type=jnp.float32)
    amax = jnp.max(a_s)
    asrc_ref[...] = jnp.concatenate(
        [a_s, jnp.full((16, 1), amax, jnp.float32)], axis=0)
    adst_ref[...] = a_d


def _tc_mid(part, wsum, b, w2, aw_s, aw_d):
    return pl.pallas_call(
        _tc_mid_body,
        out_shape=[
            jax.ShapeDtypeStruct((N, D), jnp.float32),
            jax.ShapeDtypeStruct((N, HD), jnp.float32),
            jax.ShapeDtypeStruct((N, HD), jnp.float32),
            jax.ShapeDtypeStruct((N + 16, 1), jnp.float32),
            jax.ShapeDtypeStruct((N, 1), jnp.float32),
        ],
    )(part, wsum, b, w2, aw_s, aw_d)


def _tc_post_body(part_ref, wsum_ref, b_ref, out_ref):
    unnorm = jnp.concatenate([part_ref[0, :N], part_ref[1, :N]], axis=1)
    wsum = wsum_ref[:N]
    out_ref[...] = unnorm / (wsum[:, None] + 1e-16) + b_ref[...][None, :]


def _tc_post(part, wsum, b):
    return pl.pallas_call(
        _tc_post_body,
        out_shape=jax.ShapeDtypeStruct((N, D), jnp.float32),
    )(part, wsum, b)


def _sc_body(src_hbm, dst_hbm, asrc_hbm, adst_hbm, hlo_hbm, hhi_hbm,
             part_hbm, wsum_hbm,
             src_v, dst_v, asrc_v, adst_v, alpha_v, rows_v, zvec_v,
             out_sp, wsum_sp):
    cid = lax.axis_index("c")
    sid = lax.axis_index("s")

    pltpu.sync_copy(asrc_hbm, asrc_v)
    pltpu.sync_copy(adst_hbm, adst_v)
    pltpu.sync_copy(src_hbm.at[sid], src_v)
    pltpu.sync_copy(dst_hbm.at[sid], dst_v)

    zeros16 = jnp.zeros((16,), jnp.float32)

    def _zero_rows(j, carry):
        for q in range(HD // 16):
            rows_v[j, pl.ds(q * 16, 16)] = zeros16
        return carry

    lax.fori_loop(0, ZB, _zero_rows, 0)

    def _zero_zvec(j, carry):
        zvec_v[pl.ds(j * 16, 16)] = zeros16
        return carry

    lax.fori_loop(0, ZSTRIDE // 16, _zero_zvec, 0)

    row0 = sid * ZSTRIDE
    for q in range(ZSTRIDE // ZB):
        pltpu.sync_copy(rows_v, out_sp.at[pl.ds(row0 + q * ZB, ZB)])
    pltpu.sync_copy(zvec_v, wsum_sp.at[pl.ds(row0, ZSTRIDE)])
    plsc.subcore_barrier()

    a16 = asrc_v[pl.ds(N, 16)]

    def _chunk(j, carry):
        for t in range(K // 16):
            s16 = src_v[j, pl.ds(t * 16, 16)]
            d16 = dst_v[j, pl.ds(t * 16, 16)]
            a_s = plsc.load_gather(asrc_v, [s16])
            a_d = plsc.load_gather(adst_v, [d16])
            z = a_s + a_d
            z = jnp.where(z >= 0, z, NEG * z)
            c = a_d + a16
            c = jnp.where(c >= 0, c, NEG * c)
            alpha_v[pl.ds(t * 16, 16)] = jnp.exp(z - c)

        @pl.when(cid == 0)
        def _wsum_scatter():
            pltpu.sync_copy(alpha_v, wsum_sp.at[dst_v.at[j]], add=True)

        @pl.when(cid == 0)
        def _gather_lo():
            pltpu.sync_copy(hlo_hbm.at[src_v.at[j]], rows_v)

        @pl.when(cid == 1)
        def _gather_hi():
            pltpu.sync_copy(hhi_hbm.at[src_v.at[j]], rows_v)

        for g in range(K // 16):
            e16 = alpha_v[pl.ds(g * 16, 16)]
            for lane in range(16):
                a = e16[lane]
                r = g * 16 + lane
                for q in range(HD // 16):
                    rows_v[r, pl.ds(q * 16, 16)] = (
                        rows_v[r, pl.ds(q * 16, 16)] * a)
        pltpu.sync_copy(rows_v, out_sp.at[dst_v.at[j]], add=True)
        return carry

    lax.fori_loop(0, NCH, _chunk, 0)
    plsc.subcore_barrier()

    for q in range(ZSTRIDE // ZB):
        off = row0 + q * ZB
        pltpu.sync_copy(out_sp.at[pl.ds(off, ZB)],
                        part_hbm.at[cid, pl.ds(off, ZB)])

    @pl.when(cid == 0)
    def _wsum_out():
        pltpu.sync_copy(wsum_sp.at[pl.ds(row0, ZSTRIDE)],
                        wsum_hbm.at[pl.ds(row0, ZSTRIDE)])


@functools.partial(
    pl.kernel,
    out_type=[
        jax.ShapeDtypeStruct((NC, NP, HD), jnp.float32),
        jax.ShapeDtypeStruct((NP,), jnp.float32),
    ],
    mesh=plsc.VectorSubcoreMesh(core_axis_name="c", subcore_axis_name="s",
                                num_cores=NC, num_subcores=NS),
    compiler_params=pltpu.CompilerParams(needs_layout_passes=False,
                                         use_tc_tiling_on_sc=False),
    scratch_types=[
        pltpu.VMEM((NCH, K), jnp.int32),
        pltpu.VMEM((NCH, K), jnp.int32),
        pltpu.VMEM((N + 16,), jnp.float32),
        pltpu.VMEM((N,), jnp.float32),
        pltpu.VMEM((K,), jnp.float32),
        pltpu.VMEM((K, HD), jnp.float32),
        pltpu.VMEM((ZSTRIDE,), jnp.float32),
        pltpu.VMEM_SHARED((NP, HD), jnp.float32),
        pltpu.VMEM_SHARED((NP,), jnp.float32),
    ],
)
def _sc_edge(src3, dst3, asrc, adst, hlo, hhi, part_out, wsum_out,
             src_v, dst_v, asrc_v, adst_v, alpha_v, rows_v, zvec_v,
             out_sp, wsum_sp):
    _sc_body(src3, dst3, asrc, adst, hlo, hhi, part_out, wsum_out,
             src_v, dst_v, asrc_v, adst_v, alpha_v, rows_v, zvec_v,
             out_sp, wsum_sp)


def kernel(x, edge_index, W1, att_src1, att_dst1, b1,
           W2, att_src2, att_dst2, b2):
    src3 = edge_index[0].astype(jnp.int32).reshape(NS, NCH, K)
    dst3 = edge_index[1].astype(jnp.int32).reshape(NS, NCH, K)

    hlo1, hhi1, asrc1, adst1 = _tc_pre(
        x, W1, att_src1.reshape(D, 1), att_dst1.reshape(D, 1))
    part1, wsum1 = _sc_edge(src3, dst3, asrc1.reshape(N + 16),
                            adst1.reshape(N), hlo1, hhi1)
    x1, hlo2, hhi2, asrc2, adst2 = _tc_mid(
        part1, wsum1, b1, W2, att_src2.reshape(D, 1), att_dst2.reshape(D, 1))
    part2, wsum2 = _sc_edge(src3, dst3, asrc2.reshape(N + 16),
                            adst2.reshape(N), hlo2, hhi2)
    x2 = _tc_post(part2, wsum2, b2)
    return (x1, x2)


# speedup vs baseline: 36.9743x; 1.6204x over previous
import functools

import jax
import jax.numpy as jnp
from jax import lax
from jax.experimental import pallas as pl
from jax.experimental.pallas import tpu as pltpu
from jax.experimental.pallas import tpu_sc as plsc

N = 10000
NP = 10240
E = 320000
D = 128
HD = D // 2
NEG = 0.2

NC = 2
NS = 16
EPT = E // NS
K = 80
NCH = EPT // K
NB = 4
ZB = 80
ZSTRIDE = NP // NS
SHIFT = 14
MASK = (1 << SHIFT) - 1


def _tc_pre_body(x_ref, w_ref, aw_s_ref, aw_d_ref,
                 hlo_ref, hhi_ref, asrc_ref, adst_ref):
    h = jnp.dot(x_ref[...], w_ref[...], preferred_element_type=jnp.float32)
    hlo_ref[...] = h[:, :HD]
    hhi_ref[...] = h[:, HD:]
    a_s = jnp.dot(h, aw_s_ref[...], preferred_element_type=jnp.float32)
    a_d = jnp.dot(h, aw_d_ref[...], preferred_element_type=jnp.float32)
    amax = jnp.max(a_s)
    asrc_ref[...] = jnp.concatenate(
        [a_s, jnp.full((16, 1), amax, jnp.float32)], axis=0)
    adst_ref[...] = a_d


def _tc_pre(x, w, aw_s, aw_d):
    return pl.pallas_call(
        _tc_pre_body,
        out_shape=[
            jax.ShapeDtypeStruct((N, HD), jnp.float32),
            jax.ShapeDtypeStruct((N, HD), jnp.float32),
            jax.ShapeDtypeStruct((N + 16, 1), jnp.float32),
            jax.ShapeDtypeStruct((N, 1), jnp.float32),
        ],
    )(x, w, aw_s, aw_d)


def _tc_mid_body(part_ref, wsum_ref, b_ref, w2_ref, aw_s_ref, aw_d_ref,
                 x1_ref, hlo_ref, hhi_ref, asrc_ref, adst_ref):
    unnorm = jnp.concatenate([part_ref[0, :N], part_ref[1, :N]], axis=1)
    wsum = wsum_ref[:N]
    x1 = unnorm / (wsum[:, None] + 1e-16) + b_ref[...][None, :]
    x1 = jnp.maximum(x1, 0.0)
    x1_ref[...] = x1
    h2 = jnp.dot(x1, w2_ref[...], preferred_element_type=jnp.float32)
    hlo_ref[...] = h2[:, :HD]
    hhi_ref[...] = h2[:, HD:]
    a_s = jnp.dot(h2, aw_s_ref[...], preferred_element_type=jnp.float32)
    a_d = jnp.dot(h2, aw_d_ref[...], preferred_element_type=jnp.float32)
    amax = jnp.max(a_s)
    asrc_ref[...] = jnp.concatenate(
        [a_s, jnp.full((16, 1), amax, jnp.float32)], axis=0)
    adst_ref[...] = a_d


def _tc_mid(part, wsum, b, w2, aw_s, aw_d):
    return pl.pallas_call(
        _tc_mid_body,
        out_shape=[
            jax.ShapeDtypeStruct((N, D), jnp.float32),
            jax.ShapeDtypeStruct((N, HD), jnp.float32),
            jax.ShapeDtypeStruct((N, HD), jnp.float32),
            jax.ShapeDtypeStruct((N + 16, 1), jnp.float32),
            jax.ShapeDtypeStruct((N, 1), jnp.float32),
        ],
    )(part, wsum, b, w2, aw_s, aw_d)


def _tc_post_body(part_ref, wsum_ref, b_ref, out_ref):
    unnorm = jnp.concatenate([part_ref[0, :N], part_ref[1, :N]], axis=1)
    wsum = wsum_ref[:N]
    out_ref[...] = unnorm / (wsum[:, None] + 1e-16) + b_ref[...][None, :]


def _tc_post(part, wsum, b):
    return pl.pallas_call(
        _tc_post_body,
        out_shape=jax.ShapeDtypeStruct((N, D), jnp.float32),
    )(part, wsum, b)


def _sc_body(pack_hbm, asrc_hbm, adst_hbm, hlo_hbm, hhi_hbm,
             part_hbm, wsum_hbm,
             pack_v, asrc_v, adst_v, rows, alpha, srcb, dstb, zvec_v,
             out_sp, wsum_sp, gsem, ssem, wsem):
    cid = lax.axis_index("c")
    sid = lax.axis_index("s")

    pltpu.sync_copy(asrc_hbm, asrc_v)
    pltpu.sync_copy(adst_hbm, adst_v)
    pltpu.sync_copy(pack_hbm.at[sid], pack_v)

    zeros16 = jnp.zeros((16,), jnp.float32)

    def _zero_rows(j, carry):
        for q in range(HD // 16):
            rows[0][j, pl.ds(q * 16, 16)] = zeros16
        return carry

    lax.fori_loop(0, ZB, _zero_rows, 0)

    def _zero_zvec(j, carry):
        zvec_v[pl.ds(j * 16, 16)] = zeros16
        return carry

    lax.fori_loop(0, ZSTRIDE // 16, _zero_zvec, 0)

    row0 = sid * ZSTRIDE
    for q in range(ZSTRIDE // ZB):
        pltpu.sync_copy(rows[0], out_sp.at[pl.ds(row0 + q * ZB, ZB)])
    pltpu.sync_copy(zvec_v, wsum_sp.at[pl.ds(row0, ZSTRIDE)])
    plsc.subcore_barrier()

    a16 = asrc_v[pl.ds(N, 16)]

    def unpack_alpha(j, b):
        for t in range(K // 16):
            p16 = pack_v[j, pl.ds(t * 16, 16)]
            s16 = p16 >> SHIFT
            d16 = p16 & MASK
            srcb[b][pl.ds(t * 16, 16)] = s16
            dstb[b][pl.ds(t * 16, 16)] = d16
            a_s = plsc.load_gather(asrc_v, [s16])
            a_d = plsc.load_gather(adst_v, [d16])
            z = a_s + a_d
            z = jnp.where(z >= 0, z, NEG * z)
            c = a_d + a16
            c = jnp.where(c >= 0, c, NEG * c)
            alpha[b][pl.ds(t * 16, 16)] = jnp.exp(z - c)

    def gstart(b):
        @pl.when(cid == 0)
        def _():
            pltpu.async_copy(hlo_hbm.at[srcb[b]], rows[b], gsem[b])

        @pl.when(cid == 1)
        def _():
            pltpu.async_copy(hhi_hbm.at[srcb[b]], rows[b], gsem[b])

    def gwait(b):
        pltpu.make_async_copy(hlo_hbm.at[srcb[b]], rows[b], gsem[b]).wait()

    def sstart(b):
        pltpu.async_copy(rows[b], out_sp.at[dstb[b]], ssem[b], add=True)

    def swait(b):
        pltpu.make_async_copy(rows[b], out_sp.at[dstb[b]], ssem[b]).wait()

    def wstart(b):
        @pl.when(cid == 0)
        def _():
            pltpu.async_copy(alpha[b], wsum_sp.at[dstb[b]], wsem[b], add=True)

    def wwait(b):
        @pl.when(cid == 0)
        def _():
            pltpu.make_async_copy(alpha[b], wsum_sp.at[dstb[b]],
                                  wsem[b]).wait()

    def scale(b):
        for g in range(K // 16):
            e16 = alpha[b][pl.ds(g * 16, 16)]
            for lane in range(16):
                a = e16[lane]
                r = g * 16 + lane
                for q in range(HD // 16):
                    rows[b][r, pl.ds(q * 16, 16)] = (
                        rows[b][r, pl.ds(q * 16, 16)] * a)

    for b in (0, 1):
        unpack_alpha(b, b)
        gstart(b)
        wstart(b)

    def body(q, carry):
        for b in range(NB):
            t = NB * q + b
            b2 = (b + 2) % NB

            @pl.when(t < NCH)
            def _process():
                gwait(b)
                scale(b)
                sstart(b)

            @pl.when(t >= 2)
            def _drain():
                swait(b2)
                wwait(b2)

            @pl.when(t + 2 < NCH)
            def _issue():
                unpack_alpha(t + 2, b2)
                gstart(b2)
                wstart(b2)
        return carry

    lax.fori_loop(0, (NCH + 2 + NB - 1) // NB, body, 0)
    plsc.subcore_barrier()

    for q in range(ZSTRIDE // ZB):
        off = row0 + q * ZB
        pltpu.sync_copy(out_sp.at[pl.ds(off, ZB)],
                        part_hbm.at[cid, pl.ds(off, ZB)])

    @pl.when(cid == 0)
    def _wsum_out():
        pltpu.sync_copy(wsum_sp.at[pl.ds(row0, ZSTRIDE)],
                        wsum_hbm.at[pl.ds(row0, ZSTRIDE)])


@functools.partial(
    pl.kernel,
    out_type=[
        jax.ShapeDtypeStruct((NC, NP, HD), jnp.float32),
        jax.ShapeDtypeStruct((NP,), jnp.float32),
    ],
    mesh=plsc.VectorSubcoreMesh(core_axis_name="c", subcore_axis_name="s",
                                num_cores=NC, num_subcores=NS),
    compiler_params=pltpu.CompilerParams(needs_layout_passes=False,
                                         use_tc_tiling_on_sc=False),
    scratch_types=[
        pltpu.VMEM((NCH, K), jnp.int32),
        pltpu.VMEM((N + 16,), jnp.float32),
        pltpu.VMEM((N,), jnp.float32),
    ] + [pltpu.VMEM((K, HD), jnp.float32) for _ in range(NB)]
      + [pltpu.VMEM((K,), jnp.float32) for _ in range(NB)]
      + [pltpu.VMEM((K,), jnp.int32) for _ in range(NB)]
      + [pltpu.VMEM((K,), jnp.int32) for _ in range(NB)]
      + [
        pltpu.VMEM((ZSTRIDE,), jnp.float32),
        pltpu.VMEM_SHARED((NP, HD), jnp.float32),
        pltpu.VMEM_SHARED((NP,), jnp.float32),
    ] + [pltpu.SemaphoreType.DMA for _ in range(3 * NB)],
)
def _sc_edge(pack3, asrc, adst, hlo, hhi, part_out, wsum_out,
             pack_v, asrc_v, adst_v,
             rows0, rows1, rows2, rows3,
             alpha0, alpha1, alpha2, alpha3,
             srcb0, srcb1, srcb2, srcb3,
             dstb0, dstb1, dstb2, dstb3,
             zvec_v, out_sp, wsum_sp,
             gsem0, gsem1, gsem2, gsem3,
             ssem0, ssem1, ssem2, ssem3,
             wsem0, wsem1, wsem2, wsem3):
    _sc_body(pack3, asrc, adst, hlo, hhi, part_out, wsum_out,
             pack_v, asrc_v, adst_v,
             [rows0, rows1, rows2, rows3],
             [alpha0, alpha1, alpha2, alpha3],
             [srcb0, srcb1, srcb2, srcb3],
             [dstb0, dstb1, dstb2, dstb3],
             zvec_v, out_sp, wsum_sp,
             [gsem0, gsem1, gsem2, gsem3],
             [ssem0, ssem1, ssem2, ssem3],
             [wsem0, wsem1, wsem2, wsem3])


def kernel(x, edge_index, W1, att_src1, att_dst1, b1,
           W2, att_src2, att_dst2, b2):
    src = edge_index[0].astype(jnp.int32)
    dst = edge_index[1].astype(jnp.int32)
    pack3 = ((src << SHIFT) | dst).reshape(NS, NCH, K)

    hlo1, hhi1, asrc1, adst1 = _tc_pre(
        x, W1, att_src1.reshape(D, 1), att_dst1.reshape(D, 1))
    part1, wsum1 = _sc_edge(pack3, asrc1.reshape(N + 16),
                            adst1.reshape(N), hlo1, hhi1)
    x1, hlo2, hhi2, asrc2, adst2 = _tc_mid(
        part1, wsum1, b1, W2, att_src2.reshape(D, 1), att_dst2.reshape(D, 1))
    part2, wsum2 = _sc_edge(pack3, asrc2.reshape(N + 16),
                            adst2.reshape(N), hlo2, hhi2)
    x2 = _tc_post(part2, wsum2, b2)
    return (x1, x2)


# speedup vs baseline: 44.2569x; 1.1970x over previous
import functools

import jax
import jax.numpy as jnp
from jax import lax
from jax.experimental import pallas as pl
from jax.experimental.pallas import tpu as pltpu
from jax.experimental.pallas import tpu_sc as plsc

N = 10000
NP = 10240
E = 320000
D = 128
HD = D // 2
NEG = 0.2

NC = 2
NS = 16
EPT = E // NS
K = 80
NCH = EPT // K
NB = 4
ZB = 80
ZSTRIDE = NP // NS
SHIFT = 14
MASK = (1 << SHIFT) - 1


def _tc_pre_body(x_ref, w_ref, aw_s_ref, aw_d_ref,
                 hlo_ref, hhi_ref, asrc_ref, adst_ref):
    h = jnp.dot(x_ref[...], w_ref[...], preferred_element_type=jnp.float32)
    hlo_ref[...] = h[:, :HD]
    hhi_ref[...] = h[:, HD:]
    a_s = jnp.dot(h, aw_s_ref[...], preferred_element_type=jnp.float32)
    a_d = jnp.dot(h, aw_d_ref[...], preferred_element_type=jnp.float32)
    amax = jnp.max(a_s)
    asrc_ref[...] = jnp.concatenate(
        [a_s, jnp.full((16, 1), amax, jnp.float32)], axis=0)
    adst_ref[...] = a_d


def _tc_pre(x, w, aw_s, aw_d):
    return pl.pallas_call(
        _tc_pre_body,
        out_shape=[
            jax.ShapeDtypeStruct((N, HD), jnp.float32),
            jax.ShapeDtypeStruct((N, HD), jnp.float32),
            jax.ShapeDtypeStruct((N + 16, 1), jnp.float32),
            jax.ShapeDtypeStruct((N, 1), jnp.float32),
        ],
    )(x, w, aw_s, aw_d)


def _tc_mid_body(part_ref, wsum_ref, b_ref, w2_ref, aw_s_ref, aw_d_ref,
                 x1_ref, hlo_ref, hhi_ref, asrc_ref, adst_ref):
    unnorm = jnp.concatenate([part_ref[0, :N], part_ref[1, :N]], axis=1)
    wsum = wsum_ref[:N]
    x1 = unnorm / (wsum[:, None] + 1e-16) + b_ref[...][None, :]
    x1 = jnp.maximum(x1, 0.0)
    x1_ref[...] = x1
    h2 = jnp.dot(x1, w2_ref[...], preferred_element_type=jnp.float32)
    hlo_ref[...] = h2[:, :HD]
    hhi_ref[...] = h2[:, HD:]
    a_s = jnp.dot(h2, aw_s_ref[...], preferred_element_type=jnp.float32)
    a_d = jnp.dot(h2, aw_d_ref[...], preferred_element_type=jnp.float32)
    amax = jnp.max(a_s)
    asrc_ref[...] = jnp.concatenate(
        [a_s, jnp.full((16, 1), amax, jnp.float32)], axis=0)
    adst_ref[...] = a_d


def _tc_mid(part, wsum, b, w2, aw_s, aw_d):
    return pl.pallas_call(
        _tc_mid_body,
        out_shape=[
            jax.ShapeDtypeStruct((N, D), jnp.float32),
            jax.ShapeDtypeStruct((N, HD), jnp.float32),
            jax.ShapeDtypeStruct((N, HD), jnp.float32),
            jax.ShapeDtypeStruct((N + 16, 1), jnp.float32),
            jax.ShapeDtypeStruct((N, 1), jnp.float32),
        ],
    )(part, wsum, b, w2, aw_s, aw_d)


def _tc_post_body(part_ref, wsum_ref, b_ref, out_ref):
    unnorm = jnp.concatenate([part_ref[0, :N], part_ref[1, :N]], axis=1)
    wsum = wsum_ref[:N]
    out_ref[...] = unnorm / (wsum[:, None] + 1e-16) + b_ref[...][None, :]


def _tc_post(part, wsum, b):
    return pl.pallas_call(
        _tc_post_body,
        out_shape=jax.ShapeDtypeStruct((N, D), jnp.float32),
    )(part, wsum, b)


def _sc_body(pack_hbm, asrc_hbm, adst_hbm, hlo_hbm, hhi_hbm,
             part_hbm, wsum_hbm,
             pack_v, a16_v, rows, alpha, srcb, dstb, asv, adv, zvec_v,
             asrc_sh, adst_sh, out_sp, wsum_sp,
             gsem, ssem, wsem, asem, dsem):
    cid = lax.axis_index("c")
    sid = lax.axis_index("s")

    @pl.when(sid == 0)
    def _stage_logits():
        pltpu.sync_copy(asrc_hbm, asrc_sh)
        pltpu.sync_copy(adst_hbm, adst_sh)
    pltpu.sync_copy(pack_hbm.at[sid], pack_v)

    zeros16 = jnp.zeros((16,), jnp.float32)

    def _zero_rows(j, carry):
        for q in range(HD // 16):
            rows[0][j, pl.ds(q * 16, 16)] = zeros16
        return carry

    lax.fori_loop(0, ZB, _zero_rows, 0)

    def _zero_zvec(j, carry):
        zvec_v[pl.ds(j * 16, 16)] = zeros16
        return carry

    lax.fori_loop(0, ZSTRIDE // 16, _zero_zvec, 0)

    row0 = sid * ZSTRIDE
    for q in range(ZSTRIDE // ZB):
        pltpu.sync_copy(rows[0], out_sp.at[pl.ds(row0 + q * ZB, ZB)])
    pltpu.sync_copy(zvec_v, wsum_sp.at[pl.ds(row0, ZSTRIDE)])
    plsc.subcore_barrier()

    pltpu.sync_copy(asrc_sh.at[pl.ds(N, 16)], a16_v)
    a16 = a16_v[...]

    def unpack(j, b):
        for t in range(K // 16):
            p16 = pack_v[j, pl.ds(t * 16, 16)]
            srcb[b][pl.ds(t * 16, 16)] = p16 >> SHIFT
            dstb[b][pl.ds(t * 16, 16)] = p16 & MASK

    def astart(b):
        pltpu.async_copy(asrc_sh.at[srcb[b]], asv[b], asem[b])
        pltpu.async_copy(adst_sh.at[dstb[b]], adv[b], dsem[b])

    def await_alpha(b):
        pltpu.make_async_copy(asrc_sh.at[srcb[b]], asv[b], asem[b]).wait()
        pltpu.make_async_copy(adst_sh.at[dstb[b]], adv[b], dsem[b]).wait()

    def alpha_compute(b):
        for t in range(K // 16):
            a_s = asv[b][pl.ds(t * 16, 16)]
            a_d = adv[b][pl.ds(t * 16, 16)]
            z = a_s + a_d
            z = jnp.where(z >= 0, z, NEG * z)
            c = a_d + a16
            c = jnp.where(c >= 0, c, NEG * c)
            alpha[b][pl.ds(t * 16, 16)] = jnp.exp(z - c)

    def gstart(b):
        @pl.when(cid == 0)
        def _():
            pltpu.async_copy(hlo_hbm.at[srcb[b]], rows[b], gsem[b])

        @pl.when(cid == 1)
        def _():
            pltpu.async_copy(hhi_hbm.at[srcb[b]], rows[b], gsem[b])

    def gwait(b):
        pltpu.make_async_copy(hlo_hbm.at[srcb[b]], rows[b], gsem[b]).wait()

    def sstart(b):
        pltpu.async_copy(rows[b], out_sp.at[dstb[b]], ssem[b], add=True)

    def swait(b):
        pltpu.make_async_copy(rows[b], out_sp.at[dstb[b]], ssem[b]).wait()

    def wstart(b):
        @pl.when(cid == 0)
        def _():
            pltpu.async_copy(alpha[b], wsum_sp.at[dstb[b]], wsem[b], add=True)

    def wwait(b):
        @pl.when(cid == 0)
        def _():
            pltpu.make_async_copy(alpha[b], wsum_sp.at[dstb[b]],
                                  wsem[b]).wait()

    def scale(b):
        for g in range(K // 16):
            e16 = alpha[b][pl.ds(g * 16, 16)]
            for lane in range(16):
                a = e16[lane]
                r = g * 16 + lane
                for q in range(HD // 16):
                    rows[b][r, pl.ds(q * 16, 16)] = (
                        rows[b][r, pl.ds(q * 16, 16)] * a)

    for b in (0, 1):
        unpack(b, b)
        gstart(b)
        astart(b)

    def body(q, carry):
        for b in range(NB):
            t = NB * q + b
            b2 = (b + 2) % NB

            @pl.when(t >= 2)
            def _drain():
                swait(b2)
                wwait(b2)

            @pl.when(t + 2 < NCH)
            def _issue():
                unpack(t + 2, b2)
                gstart(b2)
                astart(b2)

            @pl.when(t < NCH)
            def _process():
                await_alpha(b)
                alpha_compute(b)
                wstart(b)
                gwait(b)
                scale(b)
                sstart(b)
        return carry

    lax.fori_loop(0, (NCH + 2 + NB - 1) // NB, body, 0)
    plsc.subcore_barrier()

    for q in range(ZSTRIDE // ZB):
        off = row0 + q * ZB
        pltpu.sync_copy(out_sp.at[pl.ds(off, ZB)],
                        part_hbm.at[cid, pl.ds(off, ZB)])

    @pl.when(cid == 0)
    def _wsum_out():
        pltpu.sync_copy(wsum_sp.at[pl.ds(row0, ZSTRIDE)],
                        wsum_hbm.at[pl.ds(row0, ZSTRIDE)])


@functools.partial(
    pl.kernel,
    out_type=[
        jax.ShapeDtypeStruct((NC, NP, HD), jnp.float32),
        jax.ShapeDtypeStruct((NP,), jnp.float32),
    ],
    mesh=plsc.VectorSubcoreMesh(core_axis_name="c", subcore_axis_name="s",
                                num_cores=NC, num_subcores=NS),
    compiler_params=pltpu.CompilerParams(needs_layout_passes=False,
                                         use_tc_tiling_on_sc=False),
    scratch_types=[
        pltpu.VMEM((NCH, K), jnp.int32),
        pltpu.VMEM((16,), jnp.float32),
    ] + [pltpu.VMEM((K, HD), jnp.float32) for _ in range(NB)]
      + [pltpu.VMEM((K,), jnp.float32) for _ in range(NB)]
      + [pltpu.VMEM((K,), jnp.int32) for _ in range(NB)]
      + [pltpu.VMEM((K,), jnp.int32) for _ in range(NB)]
      + [pltpu.VMEM((K,), jnp.float32) for _ in range(NB)]
      + [pltpu.VMEM((K,), jnp.float32) for _ in range(NB)]
      + [
        pltpu.VMEM((ZSTRIDE,), jnp.float32),
        pltpu.VMEM_SHARED((N + 16,), jnp.float32),
        pltpu.VMEM_SHARED((N,), jnp.float32),
        pltpu.VMEM_SHARED((NP, HD), jnp.float32),
        pltpu.VMEM_SHARED((NP,), jnp.float32),
    ] + [pltpu.SemaphoreType.DMA for _ in range(5 * NB)],
)
def _sc_edge(pack3, asrc, adst, hlo, hhi, part_out, wsum_out,
             pack_v, a16_v,
             rows0, rows1, rows2, rows3,
             alpha0, alpha1, alpha2, alpha3,
             srcb0, srcb1, srcb2, srcb3,
             dstb0, dstb1, dstb2, dstb3,
             asv0, asv1, asv2, asv3,
             adv0, adv1, adv2, adv3,
             zvec_v, asrc_sh, adst_sh, out_sp, wsum_sp,
             gsem0, gsem1, gsem2, gsem3,
             ssem0, ssem1, ssem2, ssem3,
             wsem0, wsem1, wsem2, wsem3,
             asem0, asem1, asem2, asem3,
             dsem0, dsem1, dsem2, dsem3):
    _sc_body(pack3, asrc, adst, hlo, hhi, part_out, wsum_out,
             pack_v, a16_v,
             [rows0, rows1, rows2, rows3],
             [alpha0, alpha1, alpha2, alpha3],
             [srcb0, srcb1, srcb2, srcb3],
             [dstb0, dstb1, dstb2, dstb3],
             [asv0, asv1, asv2, asv3],
             [adv0, adv1, adv2, adv3],
             zvec_v, asrc_sh, adst_sh, out_sp, wsum_sp,
             [gsem0, gsem1, gsem2, gsem3],
             [ssem0, ssem1, ssem2, ssem3],
             [wsem0, wsem1, wsem2, wsem3],
             [asem0, asem1, asem2, asem3],
             [dsem0, dsem1, dsem2, dsem3])


def kernel(x, edge_index, W1, att_src1, att_dst1, b1,
           W2, att_src2, att_dst2, b2):
    src = edge_index[0].astype(jnp.int32)
    dst = edge_index[1].astype(jnp.int32)
    pack3 = ((src << SHIFT) | dst).reshape(NS, NCH, K)

    hlo1, hhi1, asrc1, adst1 = _tc_pre(
        x, W1, att_src1.reshape(D, 1), att_dst1.reshape(D, 1))
    part1, wsum1 = _sc_edge(pack3, asrc1.reshape(N + 16),
                            adst1.reshape(N), hlo1, hhi1)
    x1, hlo2, hhi2, asrc2, adst2 = _tc_mid(
        part1, wsum1, b1, W2, att_src2.reshape(D, 1), att_dst2.reshape(D, 1))
    part2, wsum2 = _sc_edge(pack3, asrc2.reshape(N + 16),
                            adst2.reshape(N), hlo2, hhi2)
    x2 = _tc_post(part2, wsum2, b2)
    return (x1, x2)
